# baseline (speedup 1.0000x reference)
import functools

import jax
import jax.numpy as jnp
from jax import lax
from jax.experimental import pallas as pl
from jax.experimental.pallas import tpu as pltpu
from jax.experimental.pallas import tpu_sc as plsc

L = 16
NC = 2
NS = 16
NW = NC * NS

ROWS = 16384
COLS = 200
D = 3
VOCAB = 10
RPW = ROWS // NW
CI = 128
NCH = RPW // CI
NG = CI // L
SLABS = ((0, 96), (96, 104))


def _make_kernel():
    mesh = plsc.VectorSubcoreMesh(core_axis_name="c", subcore_axis_name="s")

    @functools.partial(
        pl.kernel,
        out_type=jax.ShapeDtypeStruct((D, COLS, ROWS), jnp.float32),
        mesh=mesh,
        scratch_types=[
            [pltpu.VMEM((L,), jnp.float32) for _ in range(D)],
            [pltpu.VMEM((COLS, CI), jnp.int32) for _ in range(2)],
            [pltpu.VMEM((COLS, CI), jnp.float32) for _ in range(D)],
            [pltpu.SemaphoreType.DMA for _ in range(2)],
            [pltpu.SemaphoreType.DMA for _ in range(len(SLABS))],
            pltpu.SemaphoreType.DMA,
        ],
        compiler_params=pltpu.CompilerParams(needs_layout_passes=False),
    )
    def emb_kernel(xt_hbm, tab_hbm, out_hbm, tab_v, xb, ob, in_sem, out_sem,
                   tab_sem):
        wid = lax.axis_index("s") * NC + lax.axis_index("c")
        base = wid * RPW

        def in_slice(ch):
            return xt_hbm.at[:, pl.ds(base + ch * CI, CI)]

        def out_slab(ch, k, s):
            j0, jn = SLABS[s]
            return out_hbm.at[k, pl.ds(j0, jn), pl.ds(base + ch * CI, CI)]

        def ob_slab(k, s):
            j0, jn = SLABS[s]
            return ob[k].at[pl.ds(j0, jn), :]

        def compute_slab(b, s):
            j0, jn = SLABS[s]

            @plsc.parallel_loop(j0 * NG, (j0 + jn) * NG, unroll=8)
            def group_loop(t):
                j = t // NG
                sl = pl.ds((t - j * NG) * L, L)
                idx = xb[b][j, sl]
                for k in range(D):
                    vals = plsc.load_gather(tab_v[k], [idx])
                    ob[k][j, sl] = vals

        pltpu.async_copy(in_slice(0), xb[0], in_sem[0])
        if NCH > 1:
            pltpu.async_copy(in_slice(1), xb[1], in_sem[1])
        for k in range(D):
            pltpu.async_copy(tab_hbm.at[pl.ds(k * L, L)], tab_v[k], tab_sem)
        for k in range(D):
            pltpu.make_async_copy(
                tab_hbm.at[pl.ds(k * L, L)], tab_v[k], tab_sem
            ).wait()
        for ch in range(NCH):
            b = ch % 2
            pltpu.make_async_copy(in_slice(ch), xb[b], in_sem[b]).wait()
            for s in range(len(SLABS)):
                if ch >= 1:
                    for k in range(D):
                        pltpu.make_async_copy(
                            ob_slab(k, s), out_slab(ch - 1, k, s), out_sem[s]
                        ).wait()
                compute_slab(b, s)
                for k in range(D):
                    pltpu.async_copy(ob_slab(k, s), out_slab(ch, k, s), out_sem[s])
            if ch + 2 < NCH:
                pltpu.async_copy(in_slice(ch + 2), xb[b], in_sem[b])
        for s in range(len(SLABS)):
            for k in range(D):
                pltpu.make_async_copy(
                    ob_slab(k, s), out_slab(NCH - 1, k, s), out_sem[s]
                ).wait()

    return emb_kernel


_emb = _make_kernel()


def kernel(x, table):
    tab_p = (
        jnp.zeros((D, L), jnp.float32).at[:, :VOCAB].set(table.T).reshape(D * L)
    )
    out = _emb(x.T, tab_p)
    return out.transpose(2, 1, 0)


# speedup vs baseline: 1.0060x; 1.0060x over previous
import functools

import jax
import jax.numpy as jnp
from jax import lax
from jax.experimental import pallas as pl
from jax.experimental.pallas import tpu as pltpu
from jax.experimental.pallas import tpu_sc as plsc

L = 16
NC = 2
NS = 16
NW = NC * NS

ROWS = 16384
COLS = 200
D = 3
VOCAB = 10
RPW = ROWS // NW
CI = 128
NCH = RPW // CI
NG = CI // L
SLABS = ((0, 96), (96, 104))


def _make_kernel():
    mesh = plsc.VectorSubcoreMesh(core_axis_name="c", subcore_axis_name="s")

    @functools.partial(
        pl.kernel,
        out_type=jax.ShapeDtypeStruct((D, COLS, ROWS), jnp.float32),
        mesh=mesh,
        scratch_types=[
            [pltpu.VMEM((L,), jnp.float32) for _ in range(D)],
            [pltpu.VMEM((COLS, CI), jnp.int32) for _ in range(2)],
            [pltpu.VMEM((COLS, CI), jnp.float32) for _ in range(D)],
            [[pltpu.SemaphoreType.DMA for _ in range(len(SLABS))]
             for _ in range(2)],
            [pltpu.SemaphoreType.DMA for _ in range(len(SLABS))],
            pltpu.SemaphoreType.DMA,
        ],
        compiler_params=pltpu.CompilerParams(needs_layout_passes=False),
    )
    def emb_kernel(xt_hbm, tab_hbm, out_hbm, tab_v, xb, ob, in_sem, out_sem,
                   tab_sem):
        wid = lax.axis_index("s") * NC + lax.axis_index("c")
        base = wid * RPW

        def in_slice(ch, s):
            j0, jn = SLABS[s]
            return xt_hbm.at[pl.ds(j0, jn), pl.ds(base + ch * CI, CI)]

        def xb_slab(b, s):
            j0, jn = SLABS[s]
            return xb[b].at[pl.ds(j0, jn), :]

        def copy_in(ch, b):
            for s in range(len(SLABS)):
                pltpu.async_copy(in_slice(ch, s), xb_slab(b, s), in_sem[b][s])

        def wait_in(ch, b, s):
            pltpu.make_async_copy(in_slice(ch, s), xb_slab(b, s), in_sem[b][s]).wait()

        def out_slab(ch, k, s):
            j0, jn = SLABS[s]
            return out_hbm.at[k, pl.ds(j0, jn), pl.ds(base + ch * CI, CI)]

        def ob_slab(k, s):
            j0, jn = SLABS[s]
            return ob[k].at[pl.ds(j0, jn), :]

        def compute_slab(b, s):
            j0, jn = SLABS[s]

            @plsc.parallel_loop(j0 * NG, (j0 + jn) * NG, unroll=4)
            def group_loop(t):
                j = t // NG
                sl = pl.ds((t - j * NG) * L, L)
                idx = xb[b][j, sl]
                for k in range(D):
                    vals = plsc.load_gather(tab_v[k], [idx])
                    ob[k][j, sl] = vals

        copy_in(0, 0)
        if NCH > 1:
            copy_in(1, 1)
        for k in range(D):
            pltpu.async_copy(tab_hbm.at[pl.ds(k * L, L)], tab_v[k], tab_sem)
        for k in range(D):
            pltpu.make_async_copy(
                tab_hbm.at[pl.ds(k * L, L)], tab_v[k], tab_sem
            ).wait()
        for ch in range(NCH):
            b = ch % 2
            for s in range(len(SLABS)):
                wait_in(ch, b, s)
                if ch >= 1:
                    for k in range(D):
                        pltpu.make_async_copy(
                            ob_slab(k, s), out_slab(ch - 1, k, s), out_sem[s]
                        ).wait()
                compute_slab(b, s)
                for k in range(D):
                    pltpu.async_copy(ob_slab(k, s), out_slab(ch, k, s), out_sem[s])
            if ch + 2 < NCH:
                copy_in(ch + 2, b)
        for s in range(len(SLABS)):
            for k in range(D):
                pltpu.make_async_copy(
                    ob_slab(k, s), out_slab(NCH - 1, k, s), out_sem[s]
                ).wait()

    return emb_kernel


_emb = _make_kernel()


def kernel(x, table):
    tab_p = (
        jnp.zeros((D, L), jnp.float32).at[:, :VOCAB].set(table.T).reshape(D * L)
    )
    out = _emb(x.T, tab_p)
    return out.transpose(2, 1, 0)


# speedup vs baseline: 1.0195x; 1.0134x over previous
import functools

import jax
import jax.numpy as jnp
from jax import lax
from jax.experimental import pallas as pl
from jax.experimental.pallas import tpu as pltpu
from jax.experimental.pallas import tpu_sc as plsc

L = 16
NC = 2
NS = 16
NW = NC * NS

ROWS = 16384
COLS = 200
D = 3
VOCAB = 10
RPW = ROWS // NW
CI = 128
NCH = RPW // CI
NG = CI // L
SLABS = ((0, 96), (96, 104))


def _make_kernel():
    mesh = plsc.VectorSubcoreMesh(core_axis_name="c", subcore_axis_name="s")

    @functools.partial(
        pl.kernel,
        out_type=jax.ShapeDtypeStruct((D, COLS, ROWS), jnp.float32),
        mesh=mesh,
        scratch_types=[
            [pltpu.VMEM((L,), jnp.float32) for _ in range(D)],
            [pltpu.VMEM((COLS, CI), jnp.int32) for _ in range(2)],
            [pltpu.VMEM((COLS, CI), jnp.float32) for _ in range(D)],
            [[pltpu.SemaphoreType.DMA] for _ in range(2)],
            [pltpu.SemaphoreType.DMA for _ in range(len(SLABS))],
            pltpu.SemaphoreType.DMA,
        ],
        compiler_params=pltpu.CompilerParams(needs_layout_passes=False),
    )
    def emb_kernel(xt_hbm, tab_hbm, out_hbm, tab_v, xb, ob, in_sem, out_sem,
                   tab_sem):
        wid = lax.axis_index("s") * NC + lax.axis_index("c")
        base = wid * RPW

        def in_slice(ch):
            return xt_hbm.at[:, pl.ds(base + ch * CI, CI)]

        def copy_in(ch, b):
            pltpu.async_copy(in_slice(ch), xb[b], in_sem[b][0])

        def wait_in(ch, b):
            pltpu.make_async_copy(in_slice(ch), xb[b], in_sem[b][0]).wait()

        def out_slab(ch, k, s):
            j0, jn = SLABS[s]
            return out_hbm.at[k, pl.ds(j0, jn), pl.ds(base + ch * CI, CI)]

        def ob_slab(k, s):
            j0, jn = SLABS[s]
            return ob[k].at[pl.ds(j0, jn), :]

        def compute_slab(b, s):
            j0, jn = SLABS[s]

            @plsc.parallel_loop(j0 * NG, (j0 + jn) * NG, unroll=2)
            def group_loop(t):
                j = t // NG
                sl = pl.ds((t - j * NG) * L, L)
                idx = xb[b][j, sl]
                for k in range(D):
                    vals = plsc.load_gather(tab_v[k], [idx])
                    ob[k][j, sl] = vals

        copy_in(0, 0)
        if NCH > 1:
            copy_in(1, 1)
        for k in range(D):
            pltpu.async_copy(tab_hbm.at[pl.ds(k * L, L)], tab_v[k], tab_sem)
        for k in range(D):
            pltpu.make_async_copy(
                tab_hbm.at[pl.ds(k * L, L)], tab_v[k], tab_sem
            ).wait()
        for ch in range(NCH):
            b = ch % 2
            wait_in(ch, b)
            for s in range(len(SLABS)):
                if ch >= 1:
                    for k in range(D):
                        pltpu.make_async_copy(
                            ob_slab(k, s), out_slab(ch - 1, k, s), out_sem[s]
                        ).wait()
                compute_slab(b, s)
                for k in range(D):
                    pltpu.async_copy(ob_slab(k, s), out_slab(ch, k, s), out_sem[s])
            if ch + 2 < NCH:
                copy_in(ch + 2, b)
        for s in range(len(SLABS)):
            for k in range(D):
                pltpu.make_async_copy(
                    ob_slab(k, s), out_slab(NCH - 1, k, s), out_sem[s]
                ).wait()

    return emb_kernel


_emb = _make_kernel()


def kernel(x, table):
    tab_p = (
        jnp.zeros((D, L), jnp.float32).at[:, :VOCAB].set(table.T).reshape(D * L)
    )
    out = _emb(x.T, tab_p)
    return out.transpose(2, 1, 0)


# speedup vs baseline: 1.0302x; 1.0105x over previous
import functools

import jax
import jax.numpy as jnp
from jax import lax
from jax.experimental import pallas as pl
from jax.experimental.pallas import tpu as pltpu
from jax.experimental.pallas import tpu_sc as plsc

L = 16
NC = 2
NS = 16
NW = NC * NS

ROWS = 16384
COLS = 200
D = 3
VOCAB = 10
RPW = ROWS // NW
CI = 128
NCH = RPW // CI
NG = CI // L
SLABS = ((0, 64), (64, 64), (128, 72))


def _make_kernel():
    mesh = plsc.VectorSubcoreMesh(core_axis_name="c", subcore_axis_name="s")

    @functools.partial(
        pl.kernel,
        out_type=jax.ShapeDtypeStruct((D, COLS, ROWS), jnp.float32),
        mesh=mesh,
        scratch_types=[
            [pltpu.VMEM((L,), jnp.float32) for _ in range(D)],
            [pltpu.VMEM((COLS, CI), jnp.int32) for _ in range(2)],
            [pltpu.VMEM((COLS, CI), jnp.float32) for _ in range(D)],
            [[pltpu.SemaphoreType.DMA] for _ in range(2)],
            [pltpu.SemaphoreType.DMA for _ in range(len(SLABS))],
            pltpu.SemaphoreType.DMA,
        ],
        compiler_params=pltpu.CompilerParams(needs_layout_passes=False),
    )
    def emb_kernel(xt_hbm, tab_hbm, out_hbm, tab_v, xb, ob, in_sem, out_sem,
                   tab_sem):
        wid = lax.axis_index("s") * NC + lax.axis_index("c")
        base = wid * RPW

        def in_slice(ch):
            return xt_hbm.at[:, pl.ds(base + ch * CI, CI)]

        def copy_in(ch, b):
            pltpu.async_copy(in_slice(ch), xb[b], in_sem[b][0])

        def wait_in(ch, b):
            pltpu.make_async_copy(in_slice(ch), xb[b], in_sem[b][0]).wait()

        def out_slab(ch, k, s):
            j0, jn = SLABS[s]
            return out_hbm.at[k, pl.ds(j0, jn), pl.ds(base + ch * CI, CI)]

        def ob_slab(k, s):
            j0, jn = SLABS[s]
            return ob[k].at[pl.ds(j0, jn), :]

        def compute_slab(b, s):
            j0, jn = SLABS[s]

            @plsc.parallel_loop(j0 * NG, (j0 + jn) * NG, unroll=2)
            def group_loop(t):
                j = t // NG
                sl = pl.ds((t - j * NG) * L, L)
                idx = xb[b][j, sl]
                for k in range(D):
                    vals = plsc.load_gather(tab_v[k], [idx])
                    ob[k][j, sl] = vals

        copy_in(0, 0)
        if NCH > 1:
            copy_in(1, 1)
        for k in range(D):
            pltpu.async_copy(tab_hbm.at[pl.ds(k * L, L)], tab_v[k], tab_sem)
        for k in range(D):
            pltpu.make_async_copy(
                tab_hbm.at[pl.ds(k * L, L)], tab_v[k], tab_sem
            ).wait()
        for ch in range(NCH):
            b = ch % 2
            wait_in(ch, b)
            for s in range(len(SLABS)):
                if ch >= 1:
                    for k in range(D):
                        pltpu.make_async_copy(
                            ob_slab(k, s), out_slab(ch - 1, k, s), out_sem[s]
                        ).wait()
                compute_slab(b, s)
                for k in range(D):
                    pltpu.async_copy(ob_slab(k, s), out_slab(ch, k, s), out_sem[s])
            if ch + 2 < NCH:
                copy_in(ch + 2, b)
        for s in range(len(SLABS)):
            for k in range(D):
                pltpu.make_async_copy(
                    ob_slab(k, s), out_slab(NCH - 1, k, s), out_sem[s]
                ).wait()

    return emb_kernel


_emb = _make_kernel()


def kernel(x, table):
    tab_p = (
        jnp.zeros((D, L), jnp.float32).at[:, :VOCAB].set(table.T).reshape(D * L)
    )
    out = _emb(x.T, tab_p)
    return out.transpose(2, 1, 0)
